# baseline (speedup 1.0000x reference)
import jax
import jax.numpy as jnp
from jax.experimental import pallas as pl
from jax.experimental.pallas import tpu as pltpu

_EPS = 1e-06


def _densities_kernel(pts_ref, mns_ref, sgs_ref, out_ref):
    x = pts_ref[0]
    m_t = mns_ref[0]
    w_t = 1.0 / (_EPS + sgs_ref[0])
    rank = x.shape[1]
    acc = None
    for r in range(rank):
        xc = x[:, r:r + 1]
        mr = m_t[r:r + 1, :]
        wr = w_t[r:r + 1, :]
        d = xc - mr
        term = d * d * wr
        acc = term if acc is None else acc + term
    out_ref[0] = jnp.exp(-0.5 * acc)


def kernel(points, means, sigmas):
    b, k, l, rank = points.shape
    c = means.shape[2]
    bk = b * k
    pts = points.reshape(bk, l, rank)
    mns = means.reshape(bk, c, rank).transpose(0, 2, 1)
    sgs = sigmas.reshape(bk, c, rank).transpose(0, 2, 1)

    out = pl.pallas_call(
        _densities_kernel,
        grid=(bk,),
        in_specs=[
            pl.BlockSpec((1, l, rank), lambda i: (i, 0, 0)),
            pl.BlockSpec((1, rank, c), lambda i: (i, 0, 0)),
            pl.BlockSpec((1, rank, c), lambda i: (i, 0, 0)),
        ],
        out_specs=pl.BlockSpec((1, l, c), lambda i: (i, 0, 0)),
        out_shape=jax.ShapeDtypeStruct((bk, l, c), jnp.float32),
        compiler_params=pltpu.CompilerParams(
            dimension_semantics=("parallel",),
        ),
    )(pts, mns, sgs)
    return out.reshape(b, k, l, c)


# speedup vs baseline: 1.0060x; 1.0060x over previous
import jax
import jax.numpy as jnp
from jax.experimental import pallas as pl
from jax.experimental.pallas import tpu as pltpu

_EPS = 1e-06
_LOG2E = 1.4426950408889634


def _densities_kernel(pts_ref, mns_ref, sgs_ref, out_ref):
    x = pts_ref[0]
    m = mns_ref[0]
    w = 1.0 / (_EPS + sgs_ref[0])
    wm = w * m
    l = x.shape[0]
    b_mat = jnp.concatenate(
        [
            w * (-0.5 * _LOG2E),
            wm * _LOG2E,
            jnp.sum(wm * m, axis=0, keepdims=True) * (-0.5 * _LOG2E),
        ],
        axis=0,
    )
    a_mat = jnp.concatenate(
        [x * x, x, jnp.ones((l, 1), jnp.float32)], axis=1
    )
    prod = jnp.dot(a_mat, b_mat, preferred_element_type=jnp.float32)
    out_ref[0] = jax.lax.exp2(prod)


def kernel(points, means, sigmas):
    b, k, l, rank = points.shape
    c = means.shape[2]
    bk = b * k
    pts = points.reshape(bk, l, rank)
    mns = means.reshape(bk, c, rank).transpose(0, 2, 1)
    sgs = sigmas.reshape(bk, c, rank).transpose(0, 2, 1)

    out = pl.pallas_call(
        _densities_kernel,
        grid=(bk,),
        in_specs=[
            pl.BlockSpec((1, l, rank), lambda i: (i, 0, 0)),
            pl.BlockSpec((1, rank, c), lambda i: (i, 0, 0)),
            pl.BlockSpec((1, rank, c), lambda i: (i, 0, 0)),
        ],
        out_specs=pl.BlockSpec((1, l, c), lambda i: (i, 0, 0)),
        out_shape=jax.ShapeDtypeStruct((bk, l, c), jnp.float32),
        compiler_params=pltpu.CompilerParams(
            dimension_semantics=("parallel",),
        ),
    )(pts, mns, sgs)
    return out.reshape(b, k, l, c)
